# Initial kernel scaffold; baseline (speedup 1.0000x reference)
import jax
import jax.numpy as jnp
from jax.experimental import pallas as pl


def kernel(embeddings, centers):
    raise NotImplementedError("write your pallas kernel here")


# speedup vs baseline: 2.8221x; 2.8221x over previous
import jax
import jax.numpy as jnp
from jax.experimental import pallas as pl
from jax.experimental.pallas import tpu as pltpu

_BATCH = 16384
_K = 512
_D = 32
_ALPHA = 0.05
_BLOCK = 2048
_GRID = _BATCH // _BLOCK


def _kmeans_loss_body(emb_ref, cen_ref, out_ref):
    i = pl.program_id(0)
    e = emb_ref[...]
    c = cen_ref[...]
    dots = jax.lax.dot_general(
        e, c, (((1,), (1,)), ((), ())),
        preferred_element_type=jnp.float32,
    )
    e_sq = jnp.sum(e * e, axis=1, keepdims=True)
    c_sq = jnp.sum(c * c, axis=1)[None, :]
    d2 = e_sq - 2.0 * dots + c_sq
    row_min = jnp.min(d2, axis=1)
    part = jnp.sum(jnp.sqrt(jnp.maximum(row_min, 0.0)))

    @pl.when(i == 0)
    def _init():
        out_ref[0, 0] = 0.0

    out_ref[0, 0] += part

    @pl.when(i == _GRID - 1)
    def _finish():
        out_ref[0, 0] = out_ref[0, 0] * (_ALPHA / _BATCH)


def kernel(embeddings, centers):
    out = pl.pallas_call(
        _kmeans_loss_body,
        grid=(_GRID,),
        in_specs=[
            pl.BlockSpec((_BLOCK, _D), lambda i: (i, 0)),
            pl.BlockSpec((_K, _D), lambda i: (0, 0)),
        ],
        out_specs=pl.BlockSpec(memory_space=pltpu.SMEM),
        out_shape=jax.ShapeDtypeStruct((1, 1), jnp.float32),
    )(embeddings, centers)
    return out[0, 0]


# speedup vs baseline: 3.2912x; 1.1662x over previous
import jax
import jax.numpy as jnp
from jax.experimental import pallas as pl
from jax.experimental.pallas import tpu as pltpu

_BATCH = 16384
_K = 512
_D = 32
_ALPHA = 0.05
_BLOCK = 2048
_GRID = _BATCH // _BLOCK
_DAUG = 72


def _kmeans_loss_body(emb_ref, cen_ref, out_ref, eaug_ref, caug_ref, acc_ref):
    i = pl.program_id(0)

    @pl.when(i == 0)
    def _init():
        c = cen_ref[...]
        caug_ref[0:_D, :] = jnp.ones((_D, _K), jnp.float32)
        caug_ref[_D:2 * _D, :] = -2.0 * c.T
        caug_ref[2 * _D:2 * _D + 8, :] = jnp.broadcast_to(
            jnp.sum(c * c, axis=1)[None, :], (8, _K)
        ) * jnp.where(
            jax.lax.broadcasted_iota(jnp.int32, (8, _K), 0) == 0, 1.0, 0.0
        )
        eaug_ref[:, 2 * _D:] = jnp.where(
            jax.lax.broadcasted_iota(jnp.int32, (_BLOCK, _DAUG - 2 * _D), 1) == 0,
            1.0, 0.0,
        )
        acc_ref[...] = jnp.zeros((_BLOCK, 1), jnp.float32)

    e = emb_ref[...]
    eaug_ref[:, 0:_D] = e * e
    eaug_ref[:, _D:2 * _D] = e
    d2 = jax.lax.dot_general(
        eaug_ref[...], caug_ref[...], (((1,), (0,)), ((), ())),
        preferred_element_type=jnp.float32,
    )
    row_min = jnp.min(d2, axis=1, keepdims=True)
    acc_ref[...] += jnp.sqrt(jnp.maximum(row_min, 0.0))

    @pl.when(i == _GRID - 1)
    def _finish():
        out_ref[0, 0] = jnp.sum(acc_ref[...]) * (_ALPHA / _BATCH)


def kernel(embeddings, centers):
    out = pl.pallas_call(
        _kmeans_loss_body,
        grid=(_GRID,),
        in_specs=[
            pl.BlockSpec((_BLOCK, _D), lambda i: (i, 0)),
            pl.BlockSpec((_K, _D), lambda i: (0, 0)),
        ],
        out_specs=pl.BlockSpec(memory_space=pltpu.SMEM),
        out_shape=jax.ShapeDtypeStruct((1, 1), jnp.float32),
        scratch_shapes=[
            pltpu.VMEM((_BLOCK, _DAUG), jnp.float32),
            pltpu.VMEM((_DAUG, _K), jnp.float32),
            pltpu.VMEM((_BLOCK, 1), jnp.float32),
        ],
    )(embeddings, centers)
    return out[0, 0]


# speedup vs baseline: 3.4524x; 1.0490x over previous
import jax
import jax.numpy as jnp
from jax.experimental import pallas as pl
from jax.experimental.pallas import tpu as pltpu

_BATCH = 16384
_K = 512
_D = 32
_ALPHA = 0.05
_BLOCK = 2048
_GRID = _BATCH // _BLOCK


def _kmeans_loss_body(emb_ref, cen_ref, out_ref, w1_ref, csq_ref, acc_ref):
    i = pl.program_id(0)

    @pl.when(i == 0)
    def _init():
        c = cen_ref[...]
        w1_ref[...] = -2.0 * c
        csq_ref[...] = jnp.sum(c * c, axis=1, keepdims=True)
        acc_ref[...] = jnp.zeros((8, _BLOCK), jnp.float32)

    e = emb_ref[...]
    m1 = jax.lax.dot_general(
        w1_ref[...], e, (((1,), (1,)), ((), ())),
        preferred_element_type=jnp.float32,
    )
    esq = jax.lax.dot_general(
        jnp.ones((8, _D), jnp.float32), e * e, (((1,), (1,)), ((), ())),
        preferred_element_type=jnp.float32,
    )
    col_min = jnp.min(m1 + csq_ref[...], axis=0, keepdims=True)
    vals = jnp.sqrt(jnp.maximum(col_min + esq[0:1, :], 0.0))
    acc_ref[0:1, :] += vals

    @pl.when(i == _GRID - 1)
    def _finish():
        out_ref[0, 0] = jnp.sum(acc_ref[0:1, :]) * (_ALPHA / _BATCH)


def kernel(embeddings, centers):
    out = pl.pallas_call(
        _kmeans_loss_body,
        grid=(_GRID,),
        in_specs=[
            pl.BlockSpec((_BLOCK, _D), lambda i: (i, 0)),
            pl.BlockSpec((_K, _D), lambda i: (0, 0)),
        ],
        out_specs=pl.BlockSpec(memory_space=pltpu.SMEM),
        out_shape=jax.ShapeDtypeStruct((1, 1), jnp.float32),
        scratch_shapes=[
            pltpu.VMEM((_K, _D), jnp.float32),
            pltpu.VMEM((_K, 1), jnp.float32),
            pltpu.VMEM((8, _BLOCK), jnp.float32),
        ],
    )(embeddings, centers)
    return out[0, 0]


# speedup vs baseline: 7.2223x; 2.0920x over previous
import jax
import jax.numpy as jnp
from jax.experimental import pallas as pl
from jax.experimental.pallas import tpu as pltpu

_BATCH = 16384
_K = 512
_D = 32
_ALPHA = 0.05
_BLOCK = 2048
_GRID = _BATCH // _BLOCK
_DAUG = _D + 8


def _kmeans_loss_body(et_ref, ct_ref, out_ref, w_ref, eaug_ref, acc_ref):
    i = pl.program_id(0)

    @pl.when(i == 0)
    def _init():
        ct = ct_ref[...]
        w_ref[0:_D, :] = -2.0 * ct
        w_ref[_D:, :] = jnp.broadcast_to(
            jnp.sum(ct * ct, axis=0, keepdims=True), (8, _K)
        ) * jnp.where(
            jax.lax.broadcasted_iota(jnp.int32, (8, _K), 0) == 0, 1.0, 0.0
        )
        eaug_ref[_D:, :] = jnp.where(
            jax.lax.broadcasted_iota(jnp.int32, (8, _BLOCK), 0) == 0, 1.0, 0.0
        )
        acc_ref[...] = jnp.zeros((8, _BLOCK), jnp.float32)

    et = et_ref[...]
    eaug_ref[0:_D, :] = et
    d2t = jax.lax.dot_general(
        w_ref[...], eaug_ref[...], (((0,), (0,)), ((), ())),
        preferred_element_type=jnp.float32,
    )
    esq = jax.lax.dot_general(
        jnp.ones((_D, 8), jnp.float32), et * et, (((0,), (0,)), ((), ())),
        preferred_element_type=jnp.float32,
    )
    col_min = jnp.min(d2t, axis=0, keepdims=True)
    vals = jnp.sqrt(jnp.maximum(col_min + esq[0:1, :], 0.0))
    acc_ref[0:1, :] += vals

    @pl.when(i == _GRID - 1)
    def _finish():
        out_ref[0, 0] = jnp.sum(acc_ref[0:1, :]) * (_ALPHA / _BATCH)


def kernel(embeddings, centers):
    out = pl.pallas_call(
        _kmeans_loss_body,
        grid=(_GRID,),
        in_specs=[
            pl.BlockSpec((_D, _BLOCK), lambda i: (0, i)),
            pl.BlockSpec((_D, _K), lambda i: (0, 0)),
        ],
        out_specs=pl.BlockSpec(memory_space=pltpu.SMEM),
        out_shape=jax.ShapeDtypeStruct((1, 1), jnp.float32),
        scratch_shapes=[
            pltpu.VMEM((_DAUG, _K), jnp.float32),
            pltpu.VMEM((_DAUG, _BLOCK), jnp.float32),
            pltpu.VMEM((8, _BLOCK), jnp.float32),
        ],
    )(embeddings.T, centers.T)
    return out[0, 0]


# speedup vs baseline: 8.5238x; 1.1802x over previous
import jax
import jax.numpy as jnp
from jax.experimental import pallas as pl
from jax.experimental.pallas import tpu as pltpu

_BATCH = 16384
_K = 512
_D = 32
_ALPHA = 0.05
_BLOCK = 4096
_GRID = _BATCH // _BLOCK
_DAUG = _D + 8


def _kmeans_loss_body(et_ref, ct_ref, out_ref, w_ref, eaug_ref, acc_ref):
    i = pl.program_id(0)

    @pl.when(i == 0)
    def _init():
        ct = ct_ref[...]
        w_ref[0:_D, :] = -2.0 * ct
        w_ref[_D:, :] = jnp.broadcast_to(
            jnp.sum(ct * ct, axis=0, keepdims=True), (8, _K)
        ) * jnp.where(
            jax.lax.broadcasted_iota(jnp.int32, (8, _K), 0) == 0, 1.0, 0.0
        )
        eaug_ref[_D:, :] = jnp.where(
            jax.lax.broadcasted_iota(jnp.int32, (8, _BLOCK), 0) == 0, 1.0, 0.0
        )
        acc_ref[...] = jnp.zeros((8, _BLOCK), jnp.float32)

    et = et_ref[...]
    eaug_ref[0:_D, :] = et
    d2t = jax.lax.dot_general(
        w_ref[...], eaug_ref[...], (((0,), (0,)), ((), ())),
        preferred_element_type=jnp.float32,
    )
    esq = jax.lax.dot_general(
        jnp.ones((_D, 8), jnp.float32), et * et, (((0,), (0,)), ((), ())),
        preferred_element_type=jnp.float32,
    )
    col_min = jnp.min(d2t, axis=0, keepdims=True)
    vals = jnp.sqrt(jnp.maximum(col_min + esq[0:1, :], 0.0))
    acc_ref[0:1, :] += vals

    @pl.when(i == _GRID - 1)
    def _finish():
        out_ref[0, 0] = jnp.sum(acc_ref[0:1, :]) * (_ALPHA / _BATCH)


def kernel(embeddings, centers):
    out = pl.pallas_call(
        _kmeans_loss_body,
        grid=(_GRID,),
        in_specs=[
            pl.BlockSpec((_D, _BLOCK), lambda i: (0, i)),
            pl.BlockSpec((_D, _K), lambda i: (0, 0)),
        ],
        out_specs=pl.BlockSpec(memory_space=pltpu.SMEM),
        out_shape=jax.ShapeDtypeStruct((1, 1), jnp.float32),
        scratch_shapes=[
            pltpu.VMEM((_DAUG, _K), jnp.float32),
            pltpu.VMEM((_DAUG, _BLOCK), jnp.float32),
            pltpu.VMEM((8, _BLOCK), jnp.float32),
        ],
    )(embeddings.T, centers.T)
    return out[0, 0]


# speedup vs baseline: 8.8020x; 1.0326x over previous
import jax
import jax.numpy as jnp
from jax.experimental import pallas as pl
from jax.experimental.pallas import tpu as pltpu

_BATCH = 16384
_K = 512
_D = 32
_ALPHA = 0.05
_BLOCK = 8192
_GRID = _BATCH // _BLOCK
_DAUG = _D + 8


def _kmeans_loss_body(et_ref, ct_ref, out_ref, w_ref, eaug_ref, acc_ref):
    i = pl.program_id(0)

    @pl.when(i == 0)
    def _init():
        ct = ct_ref[...]
        w_ref[0:_D, :] = -2.0 * ct
        w_ref[_D:, :] = jnp.broadcast_to(
            jnp.sum(ct * ct, axis=0, keepdims=True), (8, _K)
        ) * jnp.where(
            jax.lax.broadcasted_iota(jnp.int32, (8, _K), 0) == 0, 1.0, 0.0
        )
        eaug_ref[_D:, :] = jnp.where(
            jax.lax.broadcasted_iota(jnp.int32, (8, _BLOCK), 0) == 0, 1.0, 0.0
        )
        acc_ref[...] = jnp.zeros((8, _BLOCK), jnp.float32)

    et = et_ref[...]
    eaug_ref[0:_D, :] = et
    d2t = jax.lax.dot_general(
        w_ref[...], eaug_ref[...], (((0,), (0,)), ((), ())),
        preferred_element_type=jnp.float32,
    )
    esq = jax.lax.dot_general(
        jnp.ones((_D, 8), jnp.float32), et * et, (((0,), (0,)), ((), ())),
        preferred_element_type=jnp.float32,
    )
    col_min = jnp.min(d2t, axis=0, keepdims=True)
    vals = jnp.sqrt(jnp.maximum(col_min + esq[0:1, :], 0.0))
    acc_ref[0:1, :] += vals

    @pl.when(i == _GRID - 1)
    def _finish():
        out_ref[0, 0] = jnp.sum(acc_ref[0:1, :]) * (_ALPHA / _BATCH)


def kernel(embeddings, centers):
    out = pl.pallas_call(
        _kmeans_loss_body,
        grid=(_GRID,),
        in_specs=[
            pl.BlockSpec((_D, _BLOCK), lambda i: (0, i)),
            pl.BlockSpec((_D, _K), lambda i: (0, 0)),
        ],
        out_specs=pl.BlockSpec(memory_space=pltpu.SMEM),
        out_shape=jax.ShapeDtypeStruct((1, 1), jnp.float32),
        scratch_shapes=[
            pltpu.VMEM((_DAUG, _K), jnp.float32),
            pltpu.VMEM((_DAUG, _BLOCK), jnp.float32),
            pltpu.VMEM((8, _BLOCK), jnp.float32),
        ],
    )(embeddings.T, centers.T)
    return out[0, 0]


# speedup vs baseline: 8.8771x; 1.0085x over previous
import jax
import jax.numpy as jnp
from jax.experimental import pallas as pl
from jax.experimental.pallas import tpu as pltpu

_BATCH = 16384
_K = 512
_D = 32
_ALPHA = 0.05
_BLOCK = 16384
_GRID = _BATCH // _BLOCK
_DAUG = _D + 8


def _kmeans_loss_body(et_ref, ct_ref, out_ref, w_ref, eaug_ref, acc_ref):
    i = pl.program_id(0)

    @pl.when(i == 0)
    def _init():
        ct = ct_ref[...]
        w_ref[0:_D, :] = -2.0 * ct
        w_ref[_D:, :] = jnp.broadcast_to(
            jnp.sum(ct * ct, axis=0, keepdims=True), (8, _K)
        ) * jnp.where(
            jax.lax.broadcasted_iota(jnp.int32, (8, _K), 0) == 0, 1.0, 0.0
        )
        eaug_ref[_D:, :] = jnp.where(
            jax.lax.broadcasted_iota(jnp.int32, (8, _BLOCK), 0) == 0, 1.0, 0.0
        )
        acc_ref[...] = jnp.zeros((8, _BLOCK), jnp.float32)

    et = et_ref[...]
    eaug_ref[0:_D, :] = et
    d2t = jax.lax.dot_general(
        w_ref[...], eaug_ref[...], (((0,), (0,)), ((), ())),
        preferred_element_type=jnp.float32,
    )
    esq = jax.lax.dot_general(
        jnp.ones((_D, 8), jnp.float32), et * et, (((0,), (0,)), ((), ())),
        preferred_element_type=jnp.float32,
    )
    col_min = jnp.min(d2t, axis=0, keepdims=True)
    vals = jnp.sqrt(jnp.maximum(col_min + esq[0:1, :], 0.0))
    acc_ref[0:1, :] += vals

    @pl.when(i == _GRID - 1)
    def _finish():
        out_ref[0, 0] = jnp.sum(acc_ref[0:1, :]) * (_ALPHA / _BATCH)


def kernel(embeddings, centers):
    out = pl.pallas_call(
        _kmeans_loss_body,
        grid=(_GRID,),
        in_specs=[
            pl.BlockSpec((_D, _BLOCK), lambda i: (0, i)),
            pl.BlockSpec((_D, _K), lambda i: (0, 0)),
        ],
        out_specs=pl.BlockSpec(memory_space=pltpu.SMEM),
        out_shape=jax.ShapeDtypeStruct((1, 1), jnp.float32),
        scratch_shapes=[
            pltpu.VMEM((_DAUG, _K), jnp.float32),
            pltpu.VMEM((_DAUG, _BLOCK), jnp.float32),
            pltpu.VMEM((8, _BLOCK), jnp.float32),
        ],
    )(embeddings.T, centers.T)
    return out[0, 0]


# speedup vs baseline: 8.9886x; 1.0126x over previous
import jax
import jax.numpy as jnp
from jax.experimental import pallas as pl
from jax.experimental.pallas import tpu as pltpu

_BATCH = 16384
_K = 512
_D = 32
_ALPHA = 0.05
_DAUG = _D + 8
_CHUNK = 2048
_NCHUNK = _BATCH // _CHUNK


def _kmeans_loss_body(et_ref, ct_ref, out_ref, w_ref, eaug_ref):
    ct = ct_ref[...]
    w_ref[0:_D, :] = -2.0 * ct
    w_ref[_D:, :] = jnp.broadcast_to(
        jnp.sum(ct * ct, axis=0, keepdims=True), (8, _K)
    ) * jnp.where(
        jax.lax.broadcasted_iota(jnp.int32, (8, _K), 0) == 0, 1.0, 0.0
    )
    eaug_ref[0:_D, :] = et_ref[...]
    eaug_ref[_D:, :] = jnp.where(
        jax.lax.broadcasted_iota(jnp.int32, (8, _BATCH), 0) == 0, 1.0, 0.0
    )

    partials = []
    for c in range(_NCHUNK):
        sl = pl.ds(c * _CHUNK, _CHUNK)
        d2t = jax.lax.dot_general(
            w_ref[...], eaug_ref[:, sl], (((0,), (0,)), ((), ())),
            preferred_element_type=jnp.float32,
        )
        et_c = et_ref[:, sl]
        esq = jax.lax.dot_general(
            jnp.ones((_D, 8), jnp.float32), et_c * et_c,
            (((0,), (0,)), ((), ())),
            preferred_element_type=jnp.float32,
        )
        col_min = jnp.min(d2t, axis=0, keepdims=True)
        vals = jnp.sqrt(jnp.maximum(col_min + esq[0:1, :], 0.0))
        partials.append(jnp.sum(vals))

    total = partials[0]
    for p in partials[1:]:
        total = total + p
    out_ref[0, 0] = total * (_ALPHA / _BATCH)


def kernel(embeddings, centers):
    out = pl.pallas_call(
        _kmeans_loss_body,
        grid=(1,),
        in_specs=[
            pl.BlockSpec((_D, _BATCH), lambda i: (0, 0)),
            pl.BlockSpec((_D, _K), lambda i: (0, 0)),
        ],
        out_specs=pl.BlockSpec(memory_space=pltpu.SMEM),
        out_shape=jax.ShapeDtypeStruct((1, 1), jnp.float32),
        scratch_shapes=[
            pltpu.VMEM((_DAUG, _K), jnp.float32),
            pltpu.VMEM((_DAUG, _BATCH), jnp.float32),
        ],
    )(embeddings.T, centers.T)
    return out[0, 0]
